# Initial kernel scaffold; baseline (speedup 1.0000x reference)
import jax
import jax.numpy as jnp
from jax.experimental import pallas as pl


def kernel(x, edge_index, W1, b1, W2, b2):
    raise NotImplementedError("write your pallas kernel here")


# speedup vs baseline: 8.4777x; 8.4777x over previous
import functools

import jax
import jax.numpy as jnp
from jax import lax
from jax.experimental import pallas as pl
from jax.experimental.pallas import tpu as pltpu
from jax.experimental.pallas import tpu_sc as plsc

N = 10000
NP = 10240
E = 320000
EP = 327680
ER = EP // 128
D = 128
NC = 2
NS = 16
STRIPE = NP // NS

_mesh = functools.partial(
    plsc.VectorSubcoreMesh, core_axis_name="c", subcore_axis_name="s",
    num_cores=NC, num_subcores=NS,
)


def _fill_buf(buf, val):
    v = jnp.full((16,), val, dtype=jnp.float32)

    @pl.loop(0, 128)
    def _(i):
        for k in range(8):
            buf[i, pl.ds(k * 16, 16)] = v


def _zero_stripe(acc, buf, s):
    @pl.loop(0, 5)
    def _(k):
        pltpu.sync_copy(buf, acc.at[pl.ds(s * STRIPE + k * 128, 128)])


def _writeback_stripe(acc, out_hbm, c, s):
    pltpu.sync_copy(
        acc.at[pl.ds(s * STRIPE, STRIPE)],
        out_hbm.at[pl.ds(c * NP + s * STRIPE, STRIPE)],
    )


def _sc_degree(dst2d):
    @functools.partial(
        pl.kernel,
        out_type=jax.ShapeDtypeStruct((2 * NP, 128), jnp.float32),
        mesh=_mesh(),
        scratch_types=[
            pltpu.VMEM((80, 128), jnp.int32),
            pltpu.VMEM((128, 128), jnp.float32),
            pltpu.VMEM_SHARED((NP, 128), jnp.float32),
        ],
    )
    def k(dst_hbm, out_hbm, idx_d, buf, acc):
        c = lax.axis_index("c")
        s = lax.axis_index("s")
        _fill_buf(buf, 0.0)
        _zero_stripe(acc, buf, s)
        _fill_buf(buf, 1.0)
        base = c * (ER // 2) + s * 80
        pltpu.sync_copy(dst_hbm.at[pl.ds(base, 80)], idx_d)
        plsc.subcore_barrier()

        @pl.loop(0, 80)
        def _(j):
            pltpu.sync_copy(buf, acc.at[idx_d.at[j]], add=True)

        plsc.subcore_barrier()
        _writeback_stripe(acc, out_hbm, c, s)

    return k(dst2d)


def _make_sc_edge_pass(rows_per_tile, src_base_fn, dst_base_fn, tab_rows):
    def call(tab, src2d, dst2d):
        return _build_sc_edge_pass(rows_per_tile, src_base_fn,
                                   dst_base_fn)(tab, src2d, dst2d)

    return call


_ICHUNK = 40


def _build_sc_edge_pass(rows_per_tile, src_base_fn, dst_base_fn):
    n_chunks = rows_per_tile // _ICHUNK

    @functools.partial(
        pl.kernel,
        out_type=jax.ShapeDtypeStruct((2 * NP, 128), jnp.float32),
        mesh=_mesh(),
        scratch_types=[
            pltpu.VMEM((_ICHUNK, 128), jnp.int32),
            pltpu.VMEM((_ICHUNK, 128), jnp.int32),
            pltpu.VMEM((128, 128), jnp.float32),
            pltpu.VMEM((128, 128), jnp.float32),
            pltpu.VMEM_SHARED((NP, 128), jnp.float32),
            pltpu.SemaphoreType.DMA,
            pltpu.SemaphoreType.DMA,
        ],
    )
    def k(tab_hbm, src_hbm, dst_hbm, out_hbm, idx_s, idx_d, buf0, buf1,
          acc, sem0, sem1):
        c = lax.axis_index("c")
        s = lax.axis_index("s")
        _fill_buf(buf0, 0.0)
        _zero_stripe(acc, buf0, s)
        plsc.subcore_barrier()
        src_base = src_base_fn(c, s)
        dst_base = dst_base_fn(c, s)

        @pl.loop(0, n_chunks)
        def _(cc):
            pltpu.sync_copy(src_hbm.at[pl.ds(src_base + cc * _ICHUNK,
                                             _ICHUNK)], idx_s)
            pltpu.sync_copy(dst_hbm.at[pl.ds(dst_base + cc * _ICHUNK,
                                             _ICHUNK)], idx_d)
            pltpu.async_copy(tab_hbm.at[idx_s.at[0]], buf0, sem0)

            @pl.loop(0, _ICHUNK, step=2)
            def _(j):
                pltpu.make_async_copy(tab_hbm.at[pl.ds(0, 128)], buf0,
                                      sem0).wait()
                pltpu.async_copy(tab_hbm.at[idx_s.at[j + 1]], buf1, sem1)
                pltpu.sync_copy(buf0, acc.at[idx_d.at[j]], add=True)
                pltpu.make_async_copy(tab_hbm.at[pl.ds(0, 128)], buf1,
                                      sem1).wait()

                @pl.when(j + 2 < _ICHUNK)
                def _():
                    pltpu.async_copy(tab_hbm.at[idx_s.at[j + 2]], buf0, sem0)

                pltpu.sync_copy(buf1, acc.at[idx_d.at[j + 1]], add=True)

        plsc.subcore_barrier()
        _writeback_stripe(acc, out_hbm, c, s)

    return k


_sc_layer1 = _make_sc_edge_pass(
    rows_per_tile=ER // NS,
    src_base_fn=lambda c, s: c * ER + s * (ER // NS),
    dst_base_fn=lambda c, s: s * (ER // NS),
    tab_rows=2 * NP,
)

_sc_layer2 = _make_sc_edge_pass(
    rows_per_tile=ER // (NC * NS),
    src_base_fn=lambda c, s: c * (ER // 2) + s * (ER // (NC * NS)),
    dst_base_fn=lambda c, s: c * (ER // 2) + s * (ER // (NC * NS)),
    tab_rows=NP,
)


_RB = 1024
_GB = NP // _RB


def _tc_matmul_a(x_pad, W1):
    def body(x_ref, w_ref, o_ref):
        o_ref[...] = jnp.dot(x_ref[...], w_ref[...],
                             preferred_element_type=jnp.float32)

    return pl.pallas_call(
        body,
        grid=(_GB,),
        in_specs=[
            pl.BlockSpec((_RB, 128), lambda i: (i, 0)),
            pl.BlockSpec((128, 256), lambda i: (0, 0)),
        ],
        out_specs=pl.BlockSpec((_RB, 256), lambda i: (i, 0)),
        out_shape=jax.ShapeDtypeStruct((NP, 256), jnp.float32),
    )(x_pad, W1)


def _tc_scale_b(h1, degp):
    def body(d0_ref, d1_ref, h1_ref, h1p_ref, dinv_ref):
        deg = d0_ref[:, :1] + d1_ref[:, :1] + 1.0
        dinv = lax.rsqrt(deg)
        dinv_ref[...] = jnp.broadcast_to(dinv, (_RB, 128))
        h1p_ref[...] = h1_ref[...] * dinv

    return pl.pallas_call(
        body,
        grid=(_GB, 2),
        in_specs=[
            pl.BlockSpec((_RB, 128), lambda i, c: (i, 0)),
            pl.BlockSpec((_RB, 128), lambda i, c: (_GB + i, 0)),
            pl.BlockSpec((_RB, 128), lambda i, c: (i, c)),
        ],
        out_specs=[
            pl.BlockSpec((_RB, 128), lambda i, c: (c * _GB + i, 0)),
            pl.BlockSpec((_RB, 128), lambda i, c: (i, 0)),
        ],
        out_shape=[
            jax.ShapeDtypeStruct((2 * NP, 128), jnp.float32),
            jax.ShapeDtypeStruct((NP, 128), jnp.float32),
        ],
    )(degp, degp, h1)


def _tc_layer2_in(s1, h1p, dinv_b, b1_2d, W2):
    def body(s1a_ref, s1b_ref, hpa_ref, hpb_ref, dinv_ref, b1_ref, w2_ref,
             o_ref):
        dinv = dinv_ref[...]
        b1v = b1_ref[...]
        m1a = jax.nn.relu(dinv * (s1a_ref[...] + hpa_ref[...]) + b1v[:, :128])
        m1b = jax.nn.relu(dinv * (s1b_ref[...] + hpb_ref[...]) + b1v[:, 128:])
        m1 = jnp.concatenate([m1a, m1b], axis=1)
        h2 = jnp.dot(m1, w2_ref[...], preferred_element_type=jnp.float32)
        o_ref[...] = dinv * h2

    return pl.pallas_call(
        body,
        grid=(_GB,),
        in_specs=[
            pl.BlockSpec((_RB, 128), lambda i: (i, 0)),
            pl.BlockSpec((_RB, 128), lambda i: (_GB + i, 0)),
            pl.BlockSpec((_RB, 128), lambda i: (i, 0)),
            pl.BlockSpec((_RB, 128), lambda i: (_GB + i, 0)),
            pl.BlockSpec((_RB, 128), lambda i: (i, 0)),
            pl.BlockSpec((1, 256), lambda i: (0, 0)),
            pl.BlockSpec((256, 128), lambda i: (0, 0)),
        ],
        out_specs=pl.BlockSpec((_RB, 128), lambda i: (i, 0)),
        out_shape=jax.ShapeDtypeStruct((NP, 128), jnp.float32),
    )(s1, s1, h1p, h1p, dinv_b, b1_2d, W2)


def _tc_final_d(s2, h2p, dinv_b, b2_2d):
    def body(s2a_ref, s2b_ref, hp_ref, dinv_ref, b2_ref, o_ref):
        pre = dinv_ref[...] * (s2a_ref[...] + s2b_ref[...] + hp_ref[...])
        o_ref[...] = jax.nn.sigmoid(pre + b2_ref[...])

    return pl.pallas_call(
        body,
        grid=(_GB,),
        in_specs=[
            pl.BlockSpec((_RB, 128), lambda i: (i, 0)),
            pl.BlockSpec((_RB, 128), lambda i: (_GB + i, 0)),
            pl.BlockSpec((_RB, 128), lambda i: (i, 0)),
            pl.BlockSpec((_RB, 128), lambda i: (i, 0)),
            pl.BlockSpec((1, 128), lambda i: (0, 0)),
        ],
        out_specs=pl.BlockSpec((_RB, 128), lambda i: (i, 0)),
        out_shape=jax.ShapeDtypeStruct((NP, 128), jnp.float32),
    )(s2, s2, h2p, dinv_b, b2_2d)


def kernel(x, edge_index, W1, b1, W2, b2):
    ei = edge_index.astype(jnp.int32)
    src = ei[0]
    dst = ei[1]
    pad = EP - E
    src_pad = jnp.concatenate([src, jnp.zeros((pad,), jnp.int32)])
    dst_pad = jnp.concatenate([dst, jnp.full((pad,), N, jnp.int32)])
    src2d = src_pad.reshape(ER, 128)
    dst2d = dst_pad.reshape(ER, 128)
    src_l1 = jnp.concatenate([src2d, src2d + NP], axis=0)

    x_pad = jnp.pad(x, ((0, NP - N), (0, 0)))
    b1_2d = b1.reshape(1, 256)
    b2_2d = b2.reshape(1, 128)

    h1 = _tc_matmul_a(x_pad, W1)
    degp = _sc_degree(dst2d)
    h1p, dinv_b = _tc_scale_b(h1, degp)
    s1 = _sc_layer1(h1p, src_l1, dst2d)
    h2p = _tc_layer2_in(s1, h1p, dinv_b, b1_2d, W2)
    s2 = _sc_layer2(h2p, src2d, dst2d)
    out = _tc_final_d(s2, h2p, dinv_b, b2_2d)
    return out[:N]


# speedup vs baseline: 21.1054x; 2.4895x over previous
import functools

import jax
import jax.numpy as jnp
from jax import lax
from jax.experimental import pallas as pl
from jax.experimental.pallas import tpu as pltpu
from jax.experimental.pallas import tpu_sc as plsc

N = 10000
NP = 10240
E = 320000
EP = 327680
ER = EP // 128
D = 128
NC = 2
NS = 16
STRIPE = NP // NS

_mesh = functools.partial(
    plsc.VectorSubcoreMesh, core_axis_name="c", subcore_axis_name="s",
    num_cores=NC, num_subcores=NS,
)


def _fill_buf(buf, val):
    v = jnp.full((16,), val, dtype=jnp.float32)

    @pl.loop(0, 128)
    def _(i):
        for k in range(8):
            buf[i, pl.ds(k * 16, 16)] = v


def _zero_stripe(acc, buf, s):
    @pl.loop(0, 5)
    def _(k):
        pltpu.sync_copy(buf, acc.at[pl.ds(s * STRIPE + k * 128, 128)])


def _writeback_stripe(acc, out_hbm, c, s):
    pltpu.sync_copy(
        acc.at[pl.ds(s * STRIPE, STRIPE)],
        out_hbm.at[pl.ds(c * NP + s * STRIPE, STRIPE)],
    )


def _sc_degree(dst2d):
    @functools.partial(
        pl.kernel,
        out_type=jax.ShapeDtypeStruct((2 * NP, 128), jnp.float32),
        mesh=_mesh(),
        scratch_types=[
            pltpu.VMEM((80, 128), jnp.int32),
            pltpu.VMEM((128, 128), jnp.float32),
            pltpu.VMEM_SHARED((NP, 128), jnp.float32),
        ],
    )
    def k(dst_hbm, out_hbm, idx_d, buf, acc):
        c = lax.axis_index("c")
        s = lax.axis_index("s")
        _fill_buf(buf, 0.0)
        _zero_stripe(acc, buf, s)
        _fill_buf(buf, 1.0)
        base = c * (ER // 2) + s * 80
        pltpu.sync_copy(dst_hbm.at[pl.ds(base, 80)], idx_d)
        plsc.subcore_barrier()

        @pl.loop(0, 80)
        def _(j):
            pltpu.sync_copy(buf, acc.at[idx_d.at[j]], add=True)

        plsc.subcore_barrier()
        _writeback_stripe(acc, out_hbm, c, s)

    return k(dst2d)


def _make_sc_edge_pass(rows_per_tile, src_base_fn, dst_base_fn, tab_rows):
    def call(tab, src2d, dst2d):
        return _build_sc_edge_pass(rows_per_tile, src_base_fn,
                                   dst_base_fn)(tab, src2d, dst2d)

    return call


_ICHUNK = 40


def _build_sc_edge_pass(rows_per_tile, src_base_fn, dst_base_fn):
    n_chunks = rows_per_tile // _ICHUNK

    @functools.partial(
        pl.kernel,
        out_type=jax.ShapeDtypeStruct((2 * NP, 128), jnp.float32),
        mesh=_mesh(),
        scratch_types=[
            pltpu.VMEM((_ICHUNK, 128), jnp.int32),
            pltpu.VMEM((_ICHUNK, 128), jnp.int32),
            pltpu.VMEM((128, 128), jnp.float32),
            pltpu.VMEM((128, 128), jnp.float32),
            pltpu.VMEM_SHARED((NP, 128), jnp.float32),
            pltpu.SemaphoreType.DMA,
            pltpu.SemaphoreType.DMA,
        ],
    )
    def k(tab_hbm, src_hbm, dst_hbm, out_hbm, idx_s, idx_d, buf0, buf1,
          acc, sem0, sem1):
        c = lax.axis_index("c")
        s = lax.axis_index("s")
        _fill_buf(buf0, 0.0)
        _zero_stripe(acc, buf0, s)
        plsc.subcore_barrier()
        src_base = src_base_fn(c, s)
        dst_base = dst_base_fn(c, s)

        @pl.loop(0, n_chunks)
        def _(cc):
            pltpu.sync_copy(src_hbm.at[pl.ds(src_base + cc * _ICHUNK,
                                             _ICHUNK)], idx_s)
            pltpu.sync_copy(dst_hbm.at[pl.ds(dst_base + cc * _ICHUNK,
                                             _ICHUNK)], idx_d)
            pltpu.async_copy(tab_hbm.at[idx_s.at[0]], buf0, sem0)

            @pl.loop(0, _ICHUNK, step=2)
            def _(j):
                pltpu.make_async_copy(tab_hbm.at[pl.ds(0, 128)], buf0,
                                      sem0).wait()
                pltpu.async_copy(tab_hbm.at[idx_s.at[j + 1]], buf1, sem1)
                pltpu.sync_copy(buf0, acc.at[idx_d.at[j]], add=True)
                pltpu.make_async_copy(tab_hbm.at[pl.ds(0, 128)], buf1,
                                      sem1).wait()

                @pl.when(j + 2 < _ICHUNK)
                def _():
                    pltpu.async_copy(tab_hbm.at[idx_s.at[j + 2]], buf0, sem0)

                pltpu.sync_copy(buf1, acc.at[idx_d.at[j + 1]], add=True)

        plsc.subcore_barrier()
        _writeback_stripe(acc, out_hbm, c, s)

    return k


_sc_layer1 = _make_sc_edge_pass(
    rows_per_tile=ER // NS,
    src_base_fn=lambda c, s: c * ER + s * (ER // NS),
    dst_base_fn=lambda c, s: s * (ER // NS),
    tab_rows=2 * NP,
)

_sc_layer2 = _make_sc_edge_pass(
    rows_per_tile=ER // (NC * NS),
    src_base_fn=lambda c, s: c * (ER // 2) + s * (ER // (NC * NS)),
    dst_base_fn=lambda c, s: c * (ER // 2) + s * (ER // (NC * NS)),
    tab_rows=NP,
)


_RB = 1024
_GB = NP // _RB


def _tc_matmul_a(x_pad, W1):
    def body(x_ref, w_ref, o_ref):
        o_ref[...] = jnp.dot(x_ref[...], w_ref[...],
                             preferred_element_type=jnp.float32)

    return pl.pallas_call(
        body,
        grid=(_GB,),
        in_specs=[
            pl.BlockSpec((_RB, 128), lambda i: (i, 0)),
            pl.BlockSpec((128, 256), lambda i: (0, 0)),
        ],
        out_specs=pl.BlockSpec((_RB, 256), lambda i: (i, 0)),
        out_shape=jax.ShapeDtypeStruct((NP, 256), jnp.float32),
    )(x_pad, W1)


def _tc_scale_b(h1, degp):
    def body(d0_ref, d1_ref, h1_ref, h1p_ref, dinv_ref):
        deg = d0_ref[:, :1] + d1_ref[:, :1] + 1.0
        dinv = lax.rsqrt(deg)
        dinv_ref[...] = jnp.broadcast_to(dinv, (_RB, 128))
        h1p_ref[...] = h1_ref[...] * dinv

    return pl.pallas_call(
        body,
        grid=(_GB, 2),
        in_specs=[
            pl.BlockSpec((_RB, 128), lambda i, c: (i, 0)),
            pl.BlockSpec((_RB, 128), lambda i, c: (_GB + i, 0)),
            pl.BlockSpec((_RB, 128), lambda i, c: (i, c)),
        ],
        out_specs=[
            pl.BlockSpec((_RB, 128), lambda i, c: (c * _GB + i, 0)),
            pl.BlockSpec((_RB, 128), lambda i, c: (i, 0)),
        ],
        out_shape=[
            jax.ShapeDtypeStruct((2 * NP, 128), jnp.float32),
            jax.ShapeDtypeStruct((NP, 128), jnp.float32),
        ],
    )(degp, degp, h1)


def _tc_layer2_in(s1, h1p, dinv_b, b1_2d, W2):
    def body(s1a_ref, s1b_ref, hpa_ref, hpb_ref, dinv_ref, b1_ref, w2_ref,
             o_ref):
        dinv = dinv_ref[...]
        b1v = b1_ref[...]
        m1a = jax.nn.relu(dinv * (s1a_ref[...] + hpa_ref[...]) + b1v[:, :128])
        m1b = jax.nn.relu(dinv * (s1b_ref[...] + hpb_ref[...]) + b1v[:, 128:])
        m1 = jnp.concatenate([m1a, m1b], axis=1)
        h2 = jnp.dot(m1, w2_ref[...], preferred_element_type=jnp.float32)
        o_ref[...] = dinv * h2

    return pl.pallas_call(
        body,
        grid=(_GB,),
        in_specs=[
            pl.BlockSpec((_RB, 128), lambda i: (i, 0)),
            pl.BlockSpec((_RB, 128), lambda i: (_GB + i, 0)),
            pl.BlockSpec((_RB, 128), lambda i: (i, 0)),
            pl.BlockSpec((_RB, 128), lambda i: (_GB + i, 0)),
            pl.BlockSpec((_RB, 128), lambda i: (i, 0)),
            pl.BlockSpec((1, 256), lambda i: (0, 0)),
            pl.BlockSpec((256, 128), lambda i: (0, 0)),
        ],
        out_specs=pl.BlockSpec((_RB, 128), lambda i: (i, 0)),
        out_shape=jax.ShapeDtypeStruct((NP, 128), jnp.float32),
    )(s1, s1, h1p, h1p, dinv_b, b1_2d, W2)


def _tc_final_d(s2, h2p, dinv_b, b2_2d):
    def body(s2a_ref, s2b_ref, hp_ref, dinv_ref, b2_ref, o_ref):
        pre = dinv_ref[...] * (s2a_ref[...] + s2b_ref[...] + hp_ref[...])
        o_ref[...] = jax.nn.sigmoid(pre + b2_ref[...])

    return pl.pallas_call(
        body,
        grid=(_GB,),
        in_specs=[
            pl.BlockSpec((_RB, 128), lambda i: (i, 0)),
            pl.BlockSpec((_RB, 128), lambda i: (_GB + i, 0)),
            pl.BlockSpec((_RB, 128), lambda i: (i, 0)),
            pl.BlockSpec((_RB, 128), lambda i: (i, 0)),
            pl.BlockSpec((1, 128), lambda i: (0, 0)),
        ],
        out_specs=pl.BlockSpec((_RB, 128), lambda i: (i, 0)),
        out_shape=jax.ShapeDtypeStruct((NP, 128), jnp.float32),
    )(s2, s2, h2p, dinv_b, b2_2d)


def kernel(x, edge_index, W1, b1, W2, b2):
    ei = edge_index.astype(jnp.int32)
    src = ei[0]
    dst = ei[1]
    pad = EP - E
    dump = N + (jnp.arange(pad, dtype=jnp.int32) % (NP - N))
    src_pad = jnp.concatenate([src, dump])
    dst_pad = jnp.concatenate([dst, dump])
    src2d = src_pad.reshape(ER, 128)
    dst2d = dst_pad.reshape(ER, 128)
    src_l1 = jnp.concatenate([src2d, src2d + NP], axis=0)

    x_pad = jnp.pad(x, ((0, NP - N), (0, 0)))
    b1_2d = b1.reshape(1, 256)
    b2_2d = b2.reshape(1, 128)

    h1 = _tc_matmul_a(x_pad, W1)
    degp = _sc_degree(dst2d)
    h1p, dinv_b = _tc_scale_b(h1, degp)
    s1 = _sc_layer1(h1p, src_l1, dst2d)
    h2p = _tc_layer2_in(s1, h1p, dinv_b, b1_2d, W2)
    s2 = _sc_layer2(h2p, src2d, dst2d)
    out = _tc_final_d(s2, h2p, dinv_b, b2_2d)
    return out[:N]
